# Initial kernel scaffold; baseline (speedup 1.0000x reference)
import jax
import jax.numpy as jnp
from jax.experimental import pallas as pl


def kernel(embeddings, scaffolds, batch):
    raise NotImplementedError("write your pallas kernel here")


# speedup vs baseline: 23.9250x; 23.9250x over previous
import jax
import jax.numpy as jnp
from jax import lax
from jax.experimental import pallas as pl

_S = 128
_WEIGHT = 0.05


def _loss_body(x_ref, s_ref, out_ref):
    x = x_ref[:]
    s = s_ref[:]
    B, D = x.shape
    iota = lax.broadcasted_iota(jnp.int32, (B, _S), 1)
    onehot = (s == iota).astype(jnp.float32)
    dn = (((0,), (0,)), ((), ()))
    sums = lax.dot_general(onehot, x, dn, preferred_element_type=jnp.float32)
    sq = lax.dot_general(onehot, x * x, dn, preferred_element_type=jnp.float32)
    counts = jnp.sum(onehot, axis=0)
    safe = jnp.maximum(counts, 1.0)
    var = (jnp.sum(sq, axis=1) - jnp.sum(sums * sums, axis=1) / safe) / (safe * D)
    mask = (counts > 1.0).astype(jnp.float32)
    total = jnp.sum(var * mask)
    nsc = jnp.sum(mask)
    loss = jnp.where(nsc > 0, _WEIGHT * total / jnp.maximum(nsc, 1.0), 0.0)
    out_ref[:] = jnp.reshape(loss, (1, 1))


def kernel(embeddings, scaffolds, batch):
    del batch
    B = embeddings.shape[0]
    s2 = scaffolds.astype(jnp.int32).reshape(B, 1)
    out = pl.pallas_call(
        _loss_body,
        out_shape=jax.ShapeDtypeStruct((1, 1), jnp.float32),
    )(embeddings, s2)
    return out[0, 0]
